# baseline (speedup 1.0000x reference)
import functools

import jax
import jax.numpy as jnp
from jax import lax
from jax.experimental import pallas as pl
from jax.experimental.pallas import tpu as pltpu
from jax.experimental.pallas import tpu_sc as plsc

B = 65536
K = 64
BC = 2048
NB = B // BC

_info = plsc.get_sparse_core_info()
_NC, _NS = _info.num_cores, _info.num_subcores
NW = _NC * _NS
BPW = B // NW


def _sc_gather(query_preds, seg_starts, seg_lens):
    mesh = plsc.VectorSubcoreMesh(core_axis_name="c", subcore_axis_name="s")

    @functools.partial(
        pl.kernel,
        mesh=mesh,
        out_type=[
            jax.ShapeDtypeStruct((B,), jnp.int32),
            jax.ShapeDtypeStruct((B,), jnp.int32),
        ],
        scratch_types=[
            pltpu.VMEM((BPW,), jnp.int32),
            pltpu.VMEM((BPW,), jnp.int32),
            pltpu.VMEM((BPW,), jnp.int32),
            pltpu.SemaphoreType.DMA,
            pltpu.SemaphoreType.DMA,
        ],
    )
    def body(qp_hbm, starts_hbm, lens_hbm, out_s_hbm, out_l_hbm,
             qp_v, s_v, l_v, sem_s, sem_l):
        wid = lax.axis_index("s") * _NC + lax.axis_index("c")
        base = wid * BPW
        pltpu.sync_copy(qp_hbm.at[pl.ds(base, BPW)], qp_v)
        cp_s = pltpu.async_copy(starts_hbm.at[qp_v], s_v, sem_s)
        cp_l = pltpu.async_copy(lens_hbm.at[qp_v], l_v, sem_l)
        cp_s.wait()
        wr_s = pltpu.async_copy(s_v, out_s_hbm.at[pl.ds(base, BPW)], sem_s)
        cp_l.wait()
        wr_l = pltpu.async_copy(l_v, out_l_hbm.at[pl.ds(base, BPW)], sem_l)
        wr_s.wait()
        wr_l.wait()

    return body(query_preds, seg_starts, seg_lens)


def _tc_item_mask_body(s_ref, l_ref, offs_ref, item_ref, mask_ref):
    s_row = s_ref[0]
    l_row = l_ref[0]
    o_col = offs_ref[:, 0:1]
    item_ref[...] = s_row + o_col
    mask_ref[...] = (o_col < l_row).astype(jnp.int8)


def _tc_item_mask(starts_r, lens_r, offs_c):
    return pl.pallas_call(
        _tc_item_mask_body,
        grid=(NB,),
        in_specs=[
            pl.BlockSpec((1, 1, BC), lambda i: (i, 0, 0)),
            pl.BlockSpec((1, 1, BC), lambda i: (i, 0, 0)),
            pl.BlockSpec((K, 128), lambda i: (0, 0)),
        ],
        out_specs=[
            pl.BlockSpec((K, BC), lambda i: (0, i)),
            pl.BlockSpec((K, BC), lambda i: (0, i)),
        ],
        out_shape=[
            jax.ShapeDtypeStruct((K, B), jnp.int32),
            jax.ShapeDtypeStruct((K, B), jnp.int8),
        ],
    )(starts_r, lens_r, offs_c)


def _tc_qidx_body(qidx_ref):
    i = pl.program_id(0)
    qidx_ref[...] = lax.broadcasted_iota(jnp.int32, (K, BC), 1) + i * BC


def _tc_qidx():
    return pl.pallas_call(
        _tc_qidx_body,
        grid=(NB,),
        out_specs=pl.BlockSpec((K, BC), lambda i: (0, i)),
        out_shape=jax.ShapeDtypeStruct((K, B), jnp.int32),
    )()


def kernel(query_preds, max_pairs, seg_starts, seg_lens):
    qidx_t = _tc_qidx()
    starts_g, lens_g = _sc_gather(query_preds, seg_starts, seg_lens)
    pad = (jnp.asarray(max_pairs, jnp.int32) - K)
    offs = jnp.arange(K, dtype=jnp.int32) + pad
    offs_c = jnp.broadcast_to(offs[:, None], (K, 128))
    item_t, mask_t = _tc_item_mask(
        starts_g.reshape(NB, 1, BC), lens_g.reshape(NB, 1, BC), offs_c)
    mask = mask_t.T.view(jnp.bool_)
    return (item_t.T, mask, qidx_t.T)


# speedup vs baseline: 1.0091x; 1.0091x over previous
import functools

import jax
import jax.numpy as jnp
from jax import lax
from jax.experimental import pallas as pl
from jax.experimental.pallas import tpu as pltpu
from jax.experimental.pallas import tpu_sc as plsc

B = 65536
K = 64
BC = 2048
NB = B // BC

_info = plsc.get_sparse_core_info()
_NC, _NS = _info.num_cores, _info.num_subcores
NW = _NC * _NS
BPW = B // NW


def _sc_gather(query_preds, seg_starts, seg_lens):
    mesh = plsc.VectorSubcoreMesh(core_axis_name="c", subcore_axis_name="s")

    @functools.partial(
        pl.kernel,
        mesh=mesh,
        out_type=[
            jax.ShapeDtypeStruct((B,), jnp.int32),
            jax.ShapeDtypeStruct((B,), jnp.int32),
        ],
        scratch_types=[
            pltpu.VMEM((BPW,), jnp.int32),
            pltpu.VMEM((BPW,), jnp.int32),
            pltpu.VMEM((BPW,), jnp.int32),
            pltpu.SemaphoreType.DMA,
            pltpu.SemaphoreType.DMA,
        ],
    )
    def body(qp_hbm, starts_hbm, lens_hbm, out_s_hbm, out_l_hbm,
             qp_v, s_v, l_v, sem_s, sem_l):
        wid = lax.axis_index("s") * _NC + lax.axis_index("c")
        base = wid * BPW
        pltpu.sync_copy(qp_hbm.at[pl.ds(base, BPW)], qp_v)
        cp_s = pltpu.async_copy(starts_hbm.at[qp_v], s_v, sem_s)
        cp_l = pltpu.async_copy(lens_hbm.at[qp_v], l_v, sem_l)
        cp_s.wait()
        wr_s = pltpu.async_copy(s_v, out_s_hbm.at[pl.ds(base, BPW)], sem_s)
        cp_l.wait()
        wr_l = pltpu.async_copy(l_v, out_l_hbm.at[pl.ds(base, BPW)], sem_l)
        wr_s.wait()
        wr_l.wait()

    return body(query_preds, seg_starts, seg_lens)


def _tc_expand_body(s_ref, l_ref, offs_ref, item_ref, mask_ref, qidx_ref):
    i = pl.program_id(0)
    s_row = s_ref[0]
    l_row = l_ref[0]
    o_col = offs_ref[:, 0:1]
    item_ref[...] = s_row + o_col
    mask_ref[...] = (o_col < l_row).astype(jnp.int8)
    qidx_ref[...] = lax.broadcasted_iota(jnp.int32, (K, BC), 1) + i * BC


def _tc_expand(starts_r, lens_r, offs_c):
    return pl.pallas_call(
        _tc_expand_body,
        grid=(NB,),
        in_specs=[
            pl.BlockSpec((1, 1, BC), lambda i: (i, 0, 0)),
            pl.BlockSpec((1, 1, BC), lambda i: (i, 0, 0)),
            pl.BlockSpec((K, 128), lambda i: (0, 0)),
        ],
        out_specs=[
            pl.BlockSpec((K, BC), lambda i: (0, i)),
            pl.BlockSpec((K, BC), lambda i: (0, i)),
            pl.BlockSpec((K, BC), lambda i: (0, i)),
        ],
        out_shape=[
            jax.ShapeDtypeStruct((K, B), jnp.int32),
            jax.ShapeDtypeStruct((K, B), jnp.int8),
            jax.ShapeDtypeStruct((K, B), jnp.int32),
        ],
    )(starts_r, lens_r, offs_c)


def kernel(query_preds, max_pairs, seg_starts, seg_lens):
    starts_g, lens_g = _sc_gather(query_preds, seg_starts, seg_lens)
    pad = (jnp.asarray(max_pairs, jnp.int32) - K)
    offs = jnp.arange(K, dtype=jnp.int32) + pad
    offs_c = jnp.broadcast_to(offs[:, None], (K, 128))
    item_t, mask_t, qidx_t = _tc_expand(
        starts_g.reshape(NB, 1, BC), lens_g.reshape(NB, 1, BC), offs_c)
    mask = mask_t.T.view(jnp.bool_)
    return (item_t.T, mask, qidx_t.T)
